# baseline (speedup 1.0000x reference)
import functools

import jax
import jax.numpy as jnp
from jax import lax
from jax.experimental import pallas as pl
from jax.experimental.pallas import tpu as pltpu
from jax.experimental.pallas import tpu_sc as plsc

BATCH = 4096
SEQ = 200
D_X = 64
D_P = 32
NUM_WORKERS = 32
ROWS = BATCH // NUM_WORKERS
NBUF = 4


def _sc_body(x_hbm, pos_hbm, out_hbm, obuf,
             i0, i1, i2, i3, o0, o1, o2, o3):
    isems = (i0, i1, i2, i3)
    osems = (o0, o1, o2, o3)
    c = lax.axis_index("c")
    s = lax.axis_index("s")
    base = (s * 2 + c) * ROWS

    for k in range(NBUF):
        pltpu.sync_copy(pos_hbm, obuf.at[k, :, D_X:])

    in_copies = [None] * ROWS
    out_copies = [None] * ROWS

    def start_in(r):
        k = r % NBUF
        in_copies[r] = pltpu.async_copy(
            x_hbm.at[base + r], obuf.at[k, :, 0:D_X], isems[k])

    def start_out(r):
        k = r % NBUF
        out_copies[r] = pltpu.async_copy(
            obuf.at[k], out_hbm.at[base + r], osems[k])

    for r in range(NBUF):
        start_in(r)
    for r in range(ROWS):
        in_copies[r].wait()
        start_out(r)
        nr = r + 1
        if NBUF <= nr < ROWS:
            out_copies[nr - NBUF].wait()
            start_in(nr)
    for r in range(ROWS - (NBUF - 1), ROWS):
        out_copies[r].wait()


@functools.partial(jax.jit, donate_argnums=())
def kernel(x, pos_table):
    mesh = plsc.VectorSubcoreMesh(core_axis_name="c", subcore_axis_name="s")
    run = pl.kernel(
        _sc_body,
        mesh=mesh,
        out_type=jax.ShapeDtypeStruct((BATCH, SEQ, D_X + D_P), jnp.float32),
        scratch_types=(
            [pltpu.VMEM((NBUF, SEQ, D_X + D_P), jnp.float32)]
            + [pltpu.SemaphoreType.DMA] * (2 * NBUF)
        ),
        compiler_params=pltpu.CompilerParams(use_tc_tiling_on_sc=False),
    )
    return run(x, pos_table)
